# baseline (speedup 1.0000x reference)
import functools

import jax
import jax.numpy as jnp
from jax import lax
from jax.experimental import pallas as pl

_BF = jnp.bfloat16


def _dot(a, b):
    return lax.dot_general(a, b, (((1,), (0,)), ((), ())),
                           preferred_element_type=jnp.float32)


def _enc_vq_body(pT_ref, w1T_ref, b1c_ref, w2T_ref, b2c_ref, cbm2_ref,
                 c2c_ref, cbT_ref, idx_ref, quant_ref, loss_ref, *, n_codes):
    h1 = _dot(w1T_ref[...], pT_ref[...]) + b1c_ref[...]
    h1 = jnp.maximum(h1, 0.0)
    z = _dot(w2T_ref[...], h1.astype(_BF)) + b2c_ref[...]
    u = _dot(cbm2_ref[...], z.astype(_BF)) + c2c_ref[...]
    umin = jnp.min(u, axis=0, keepdims=True)
    iota = lax.broadcasted_iota(jnp.int32, u.shape, 0).astype(jnp.float32)
    idxc = jnp.min(jnp.where(u <= umin, iota, float(n_codes)),
                   axis=0, keepdims=True)
    onehot = (iota == idxc).astype(_BF)
    quantT = _dot(cbT_ref[...], onehot)

    idx_ref[...] = idxc.astype(jnp.int32)[None]
    quant_ref[...] = quantT.astype(_BF).T
    z2 = jnp.sum(z * z, axis=0, keepdims=True)
    part = jnp.sum(umin + z2)

    @pl.when(pl.program_id(0) == 0)
    def _():
        loss_ref[...] = jnp.zeros_like(loss_ref)

    loss_ref[...] += part


def _dec_body(q_ref, wp_ref, b1_ref, w2_ref, b2_ref, out_ref, *, rb, wq, hd):
    c = pl.program_id(1)
    r0 = c * rb
    for par, (a, b2) in enumerate(((0, 0), (0, 1), (1, 0), (1, 1))):
        taps = [q_ref[0, pl.ds(r0 + a + ty, rb), pl.ds(b2 + tx, wq), :]
                for ty in (0, 1) for tx in (0, 1)]
        p = jnp.concatenate(taps, axis=-1).reshape(rb * wq, 4 * taps[0].shape[-1])
        h = _dot(p, wp_ref[par]) + b1_ref[...]
        h = jnp.maximum(h, 0.0)
        o = lax.dot_general(w2_ref[...], h.astype(_BF), (((1,), (1,)), ((), ())),
                            preferred_element_type=jnp.float32)
        out_ref[par, 0, 0] = jax.nn.sigmoid(o + b2_ref[...])


def kernel(x, we1, be1, we2, be2, codebook, wd1, bd1, wd2, bd2):
    B, C, H, W = x.shape
    hd = we1.shape[0]
    D = we2.shape[0]
    K = codebook.shape[0]
    Hq, Wq = H // 2, W // 2
    N = B * Hq * Wq
    f32 = jnp.float32

    xb = x.astype(_BF)
    xp = jnp.pad(xb, ((0, 0), (0, 0), (1, 1), (1, 1)))
    taps = [lax.slice(xp, (0, 0, ky, kx),
                      (B, C, ky + 2 * Hq - 1, kx + 2 * Wq - 1), (1, 1, 2, 2))
            for ky in range(4) for kx in range(4)]
    pT = jnp.concatenate(taps, axis=1)
    pT = pT.transpose(1, 0, 2, 3).reshape(16 * C, N)

    w1T = we1.transpose(0, 2, 3, 1).reshape(hd, 16 * C).astype(_BF)
    w2T = we2[:, :, 0, 0].astype(_BF)
    cbm2 = (-2.0 * codebook).astype(_BF)
    c2c = jnp.sum(codebook * codebook, axis=1)[:, None]
    cbT_bf = codebook.T.astype(_BF)

    BM = next(bm for bm in (1536, 1024, 512, 256, 128) if N % bm == 0)
    grid_a = N // BM
    full = lambda shape: pl.BlockSpec(shape, lambda i: (0,) * len(shape))
    idx2, quant, loss_sum = pl.pallas_call(
        functools.partial(_enc_vq_body, n_codes=K),
        grid=(grid_a,),
        in_specs=[
            pl.BlockSpec((16 * C, BM), lambda i: (0, i)),
            full((hd, 16 * C)), full((hd, 1)), full((D, hd)), full((D, 1)),
            full((K, D)), full((K, 1)), full((D, K)),
        ],
        out_specs=[
            pl.BlockSpec((1, 1, BM), lambda i: (i, 0, 0)),
            pl.BlockSpec((BM, D), lambda i: (i, 0)),
            pl.BlockSpec((1, 1), lambda i: (0, 0)),
        ],
        out_shape=[
            jax.ShapeDtypeStruct((grid_a, 1, BM), jnp.int32),
            jax.ShapeDtypeStruct((N, D), _BF),
            jax.ShapeDtypeStruct((1, 1), f32),
        ],
    )(pT, w1T, be1[:, None], w2T, be2[:, None], cbm2, c2c, cbT_bf)

    idx = idx2.reshape(B, Hq, Wq)
    loss = loss_sum[0, 0] * (2.0 / (N * D))

    qp = jnp.pad(quant.reshape(B, Hq, Wq, D),
                 ((0, 0), (1, 1), (1, 1), (0, 0)))
    wt = wd1.transpose(2, 3, 1, 0)
    wpar = jnp.stack([wt[a::2, b2::2].reshape(4 * D, hd)
                      for (a, b2) in ((0, 0), (0, 1), (1, 0), (1, 1))]).astype(_BF)
    w2d = wd2[:, :, 0, 0].astype(_BF)

    RB = 16
    nchunk = Hq // RB
    out5 = pl.pallas_call(
        functools.partial(_dec_body, rb=RB, wq=Wq, hd=hd),
        grid=(B, nchunk),
        in_specs=[
            pl.BlockSpec((1, Hq + 2, Wq + 2, D), lambda b, c: (b, 0, 0, 0)),
            pl.BlockSpec((4, 4 * D, hd), lambda b, c: (0, 0, 0)),
            pl.BlockSpec((1, hd), lambda b, c: (0, 0)),
            pl.BlockSpec((3, hd), lambda b, c: (0, 0)),
            pl.BlockSpec((3, 1), lambda b, c: (0, 0)),
        ],
        out_specs=pl.BlockSpec((4, 1, 1, 3, RB * Wq),
                               lambda b, c: (0, b, c, 0, 0)),
        out_shape=jax.ShapeDtypeStruct((4, B, nchunk, 3, RB * Wq), f32),
    )(qp, wpar, bd1[None, :], w2d, bd2[:, None])

    out6 = out5.reshape(2, 2, B, nchunk, 3, RB, Wq)
    recon = out6.transpose(2, 4, 3, 5, 0, 6, 1).reshape(B, 3, H, W)
    return (recon, loss, idx)


# speedup vs baseline: 1.7021x; 1.7021x over previous
import functools

import jax
import jax.numpy as jnp
from jax import lax
from jax.experimental import pallas as pl

_BF = jnp.bfloat16


def _dot(a, b):
    return lax.dot_general(a, b, (((1,), (0,)), ((), ())),
                           preferred_element_type=jnp.float32)


def _deint_body(x_ref, s_ref, e_ref):
    e_ref[0, 0] = _dot(x_ref[0, 0], s_ref[...]).astype(_BF)


def _enc_vq_body(pT_ref, w1T_ref, b1c_ref, w2T_ref, b2c_ref, cbm2_ref,
                 c2c_ref, cbT_ref, idx_ref, quant_ref, loss_ref, *, n_codes):
    h1 = _dot(w1T_ref[...], pT_ref[...]) + b1c_ref[...]
    h1 = jnp.maximum(h1, 0.0)
    z = _dot(w2T_ref[...], h1.astype(_BF)) + b2c_ref[...]
    u = _dot(cbm2_ref[...], z.astype(_BF)) + c2c_ref[...]
    umin = jnp.min(u, axis=0, keepdims=True)
    iota = lax.broadcasted_iota(jnp.int32, u.shape, 0).astype(jnp.float32)
    idxc = jnp.min(jnp.where(u <= umin, iota, float(n_codes)),
                   axis=0, keepdims=True)
    onehot = (iota == idxc).astype(_BF)
    quantT = _dot(cbT_ref[...], onehot)

    idx_ref[...] = idxc.astype(jnp.int32)[None]
    quant_ref[...] = quantT.astype(_BF).T
    z2 = jnp.sum(z * z, axis=0, keepdims=True)
    part = jnp.sum(umin + z2)

    @pl.when(pl.program_id(0) == 0)
    def _():
        loss_ref[...] = jnp.zeros_like(loss_ref)

    loss_ref[...] += part


def _dec_body(q_ref, wp_ref, b1_ref, w2_ref, b2_ref, out_ref, *, rb, wq, hd):
    c = pl.program_id(1)
    r0 = c * rb
    for par, (a, b2) in enumerate(((0, 0), (0, 1), (1, 0), (1, 1))):
        taps = [q_ref[0, pl.ds(r0 + a + ty, rb), pl.ds(b2 + tx, wq), :]
                for ty in (0, 1) for tx in (0, 1)]
        p = jnp.concatenate(taps, axis=-1).reshape(rb * wq, 4 * taps[0].shape[-1])
        h = _dot(p, wp_ref[par]) + b1_ref[...]
        h = jnp.maximum(h, 0.0)
        o = lax.dot_general(w2_ref[...], h.astype(_BF), (((1,), (1,)), ((), ())),
                            preferred_element_type=jnp.float32)
        out_ref[par, 0, 0] = jax.nn.sigmoid(o + b2_ref[...])


def kernel(x, we1, be1, we2, be2, codebook, wd1, bd1, wd2, bd2):
    B, C, H, W = x.shape
    hd = we1.shape[0]
    D = we2.shape[0]
    K = codebook.shape[0]
    Hq, Wq = H // 2, W // 2
    N = B * Hq * Wq
    f32 = jnp.float32

    xb = x.astype(_BF)
    xp = jnp.pad(xb, ((0, 0), (0, 0), (1, 1), (1, 1)))
    Hp, Wp = H + 2, W + 2
    ncol = Wq + 1
    half = ((ncol + 127) // 128) * 128
    m = jnp.arange(2 * half)
    tgt = jnp.where(m < ncol, 2 * m,
                    jnp.where((m >= half) & (m < half + ncol),
                              2 * (m - half) + 1, -1))
    sel = (jnp.arange(Wp)[:, None] == tgt[None, :]).astype(_BF)
    e = pl.pallas_call(
        _deint_body,
        grid=(B, C),
        in_specs=[pl.BlockSpec((1, 1, Hp, Wp), lambda b, c: (b, c, 0, 0)),
                  pl.BlockSpec((Wp, 2 * half), lambda b, c: (0, 0))],
        out_specs=pl.BlockSpec((1, 1, Hp, 2 * half), lambda b, c: (b, c, 0, 0)),
        out_shape=jax.ShapeDtypeStruct((B, C, Hp, 2 * half), _BF),
    )(xp, sel)
    taps = [lax.slice(e, (0, 0, ky, (kx % 2) * half + kx // 2),
                      (B, C, ky + 2 * Hq - 1, (kx % 2) * half + kx // 2 + Wq),
                      (1, 1, 2, 1))
            for ky in range(4) for kx in range(4)]
    pT = jnp.stack(taps, axis=0)
    pT = pT.transpose(0, 2, 1, 3, 4).reshape(16 * C, N)

    w1T = we1.transpose(0, 2, 3, 1).reshape(hd, 16 * C).astype(_BF)
    w2T = we2[:, :, 0, 0].astype(_BF)
    cbm2 = (-2.0 * codebook).astype(_BF)
    c2c = jnp.sum(codebook * codebook, axis=1)[:, None]
    cbT_bf = codebook.T.astype(_BF)

    BM = next(bm for bm in (1536, 1024, 512, 256, 128) if N % bm == 0)
    grid_a = N // BM
    full = lambda shape: pl.BlockSpec(shape, lambda i: (0,) * len(shape))
    idx2, quant, loss_sum = pl.pallas_call(
        functools.partial(_enc_vq_body, n_codes=K),
        grid=(grid_a,),
        in_specs=[
            pl.BlockSpec((16 * C, BM), lambda i: (0, i)),
            full((hd, 16 * C)), full((hd, 1)), full((D, hd)), full((D, 1)),
            full((K, D)), full((K, 1)), full((D, K)),
        ],
        out_specs=[
            pl.BlockSpec((1, 1, BM), lambda i: (i, 0, 0)),
            pl.BlockSpec((BM, D), lambda i: (i, 0)),
            pl.BlockSpec((1, 1), lambda i: (0, 0)),
        ],
        out_shape=[
            jax.ShapeDtypeStruct((grid_a, 1, BM), jnp.int32),
            jax.ShapeDtypeStruct((N, D), _BF),
            jax.ShapeDtypeStruct((1, 1), f32),
        ],
    )(pT, w1T, be1[:, None], w2T, be2[:, None], cbm2, c2c, cbT_bf)

    idx = idx2.reshape(B, Hq, Wq)
    loss = loss_sum[0, 0] * (2.0 / (N * D))

    qp = jnp.pad(quant.reshape(B, Hq, Wq, D),
                 ((0, 0), (1, 1), (1, 1), (0, 0)))
    wt = wd1.transpose(2, 3, 1, 0)
    wpar = jnp.stack([wt[a::2, b2::2].reshape(4 * D, hd)
                      for (a, b2) in ((0, 0), (0, 1), (1, 0), (1, 1))]).astype(_BF)
    w2d = wd2[:, :, 0, 0].astype(_BF)

    RB = 16
    nchunk = Hq // RB
    out5 = pl.pallas_call(
        functools.partial(_dec_body, rb=RB, wq=Wq, hd=hd),
        grid=(B, nchunk),
        in_specs=[
            pl.BlockSpec((1, Hq + 2, Wq + 2, D), lambda b, c: (b, 0, 0, 0)),
            pl.BlockSpec((4, 4 * D, hd), lambda b, c: (0, 0, 0)),
            pl.BlockSpec((1, hd), lambda b, c: (0, 0)),
            pl.BlockSpec((3, hd), lambda b, c: (0, 0)),
            pl.BlockSpec((3, 1), lambda b, c: (0, 0)),
        ],
        out_specs=pl.BlockSpec((4, 1, 1, 3, RB * Wq),
                               lambda b, c: (0, b, c, 0, 0)),
        out_shape=jax.ShapeDtypeStruct((4, B, nchunk, 3, RB * Wq), f32),
    )(qp, wpar, bd1[None, :], w2d, bd2[:, None])

    out6 = out5.reshape(2, 2, B, nchunk, 3, RB, Wq)
    recon = out6.transpose(2, 4, 3, 5, 0, 6, 1).reshape(B, 3, H, W)
    return (recon, loss, idx)


# speedup vs baseline: 2.0746x; 1.2189x over previous
import functools

import jax
import jax.numpy as jnp
from jax import lax
from jax.experimental import pallas as pl

_BF = jnp.bfloat16


def _dot(a, b):
    return lax.dot_general(a, b, (((1,), (0,)), ((), ())),
                           preferred_element_type=jnp.float32)


def _deint_body(x_ref, sc_ref, sr_ref, e_ref):
    t = _dot(x_ref[0, 0], sc_ref[...]).astype(_BF)
    e_ref[0, 0] = _dot(sr_ref[...], t).astype(_BF)


def _enc_vq_body(pT_ref, w1T_ref, b1c_ref, w2T_ref, b2c_ref, cbm2_ref,
                 c2c_ref, cbT_ref, idx_ref, quant_ref, loss_ref, *, n_codes):
    h1 = _dot(w1T_ref[...], pT_ref[...]) + b1c_ref[...]
    h1 = jnp.maximum(h1, 0.0)
    z = _dot(w2T_ref[...], h1.astype(_BF)) + b2c_ref[...]
    u = _dot(cbm2_ref[...], z.astype(_BF)) + c2c_ref[...]
    umin = jnp.min(u, axis=0, keepdims=True)
    iota = lax.broadcasted_iota(jnp.int32, u.shape, 0).astype(jnp.float32)
    idxc = jnp.min(jnp.where(u <= umin, iota, float(n_codes)),
                   axis=0, keepdims=True)
    onehot = (iota == idxc).astype(_BF)
    quantT = _dot(cbT_ref[...], onehot)

    idx_ref[...] = idxc.astype(jnp.int32)[None]
    quant_ref[...] = quantT.astype(_BF).T
    z2 = jnp.sum(z * z, axis=0, keepdims=True)
    part = jnp.sum(umin + z2)

    @pl.when(pl.program_id(0) == 0)
    def _():
        loss_ref[...] = jnp.zeros_like(loss_ref)

    loss_ref[...] += part


def _dec_body(q_ref, wp_ref, b1_ref, w2_ref, b2_ref, out_ref, *, rb, wq, hd):
    c = pl.program_id(1)
    r0 = c * rb
    for par, (a, b2) in enumerate(((0, 0), (0, 1), (1, 0), (1, 1))):
        taps = [q_ref[0, pl.ds(r0 + a + ty, rb), pl.ds(b2 + tx, wq), :]
                for ty in (0, 1) for tx in (0, 1)]
        p = jnp.concatenate(taps, axis=-1).reshape(rb * wq, 4 * taps[0].shape[-1])
        h = _dot(p, wp_ref[par]) + b1_ref[...]
        h = jnp.maximum(h, 0.0)
        o = lax.dot_general(w2_ref[...], h.astype(_BF), (((1,), (1,)), ((), ())),
                            preferred_element_type=jnp.float32)
        out_ref[par, 0, 0] = jax.nn.sigmoid(o + b2_ref[...])


def kernel(x, we1, be1, we2, be2, codebook, wd1, bd1, wd2, bd2):
    B, C, H, W = x.shape
    hd = we1.shape[0]
    D = we2.shape[0]
    K = codebook.shape[0]
    Hq, Wq = H // 2, W // 2
    N = B * Hq * Wq
    f32 = jnp.float32

    xb = x.astype(_BF)
    xp = jnp.pad(xb, ((0, 0), (0, 0), (1, 1), (1, 1)))
    Hp, Wp = H + 2, W + 2
    ncol = Wq + 1
    half = ((ncol + 127) // 128) * 128
    rhalf = ((ncol + 7) // 8) * 8
    m = jnp.arange(2 * half)
    tgt = jnp.where(m < ncol, 2 * m,
                    jnp.where((m >= half) & (m < half + ncol),
                              2 * (m - half) + 1, -1))
    sel = (jnp.arange(Wp)[:, None] == tgt[None, :]).astype(_BF)
    r = jnp.arange(2 * rhalf)
    rtgt = jnp.where(r < ncol, 2 * r,
                     jnp.where((r >= rhalf) & (r < rhalf + ncol),
                               2 * (r - rhalf) + 1, -1))
    rsel = (rtgt[:, None] == jnp.arange(Hp)[None, :]).astype(_BF)
    e = pl.pallas_call(
        _deint_body,
        grid=(B, C),
        in_specs=[pl.BlockSpec((1, 1, Hp, Wp), lambda b, c: (b, c, 0, 0)),
                  pl.BlockSpec((Wp, 2 * half), lambda b, c: (0, 0)),
                  pl.BlockSpec((2 * rhalf, Hp), lambda b, c: (0, 0))],
        out_specs=pl.BlockSpec((1, 1, 2 * rhalf, 2 * half),
                               lambda b, c: (b, c, 0, 0)),
        out_shape=jax.ShapeDtypeStruct((B, C, 2 * rhalf, 2 * half), _BF),
    )(xp, sel, rsel)
    taps = [lax.slice(e, (0, 0, (ky % 2) * rhalf + ky // 2,
                          (kx % 2) * half + kx // 2),
                      (B, C, (ky % 2) * rhalf + ky // 2 + Hq,
                       (kx % 2) * half + kx // 2 + Wq),
                      (1, 1, 1, 1))
            for ky in range(4) for kx in range(4)]
    pT = jnp.stack(taps, axis=0)
    pT = pT.transpose(0, 2, 1, 3, 4).reshape(16 * C, N)

    w1T = we1.transpose(0, 2, 3, 1).reshape(hd, 16 * C).astype(_BF)
    w2T = we2[:, :, 0, 0].astype(_BF)
    cbm2 = (-2.0 * codebook).astype(_BF)
    c2c = jnp.sum(codebook * codebook, axis=1)[:, None]
    cbT_bf = codebook.T.astype(_BF)

    BM = next(bm for bm in (1536, 1024, 512, 256, 128) if N % bm == 0)
    grid_a = N // BM
    full = lambda shape: pl.BlockSpec(shape, lambda i: (0,) * len(shape))
    idx2, quant, loss_sum = pl.pallas_call(
        functools.partial(_enc_vq_body, n_codes=K),
        grid=(grid_a,),
        in_specs=[
            pl.BlockSpec((16 * C, BM), lambda i: (0, i)),
            full((hd, 16 * C)), full((hd, 1)), full((D, hd)), full((D, 1)),
            full((K, D)), full((K, 1)), full((D, K)),
        ],
        out_specs=[
            pl.BlockSpec((1, 1, BM), lambda i: (i, 0, 0)),
            pl.BlockSpec((BM, D), lambda i: (i, 0)),
            pl.BlockSpec((1, 1), lambda i: (0, 0)),
        ],
        out_shape=[
            jax.ShapeDtypeStruct((grid_a, 1, BM), jnp.int32),
            jax.ShapeDtypeStruct((N, D), _BF),
            jax.ShapeDtypeStruct((1, 1), f32),
        ],
    )(pT, w1T, be1[:, None], w2T, be2[:, None], cbm2, c2c, cbT_bf)

    idx = idx2.reshape(B, Hq, Wq)
    loss = loss_sum[0, 0] * (2.0 / (N * D))

    qp = jnp.pad(quant.reshape(B, Hq, Wq, D),
                 ((0, 0), (1, 1), (1, 1), (0, 0)))
    wt = wd1.transpose(2, 3, 1, 0)
    wpar = jnp.stack([wt[a::2, b2::2].reshape(4 * D, hd)
                      for (a, b2) in ((0, 0), (0, 1), (1, 0), (1, 1))]).astype(_BF)
    w2d = wd2[:, :, 0, 0].astype(_BF)

    RB = 16
    nchunk = Hq // RB
    out5 = pl.pallas_call(
        functools.partial(_dec_body, rb=RB, wq=Wq, hd=hd),
        grid=(B, nchunk),
        in_specs=[
            pl.BlockSpec((1, Hq + 2, Wq + 2, D), lambda b, c: (b, 0, 0, 0)),
            pl.BlockSpec((4, 4 * D, hd), lambda b, c: (0, 0, 0)),
            pl.BlockSpec((1, hd), lambda b, c: (0, 0)),
            pl.BlockSpec((3, hd), lambda b, c: (0, 0)),
            pl.BlockSpec((3, 1), lambda b, c: (0, 0)),
        ],
        out_specs=pl.BlockSpec((4, 1, 1, 3, RB * Wq),
                               lambda b, c: (0, b, c, 0, 0)),
        out_shape=jax.ShapeDtypeStruct((4, B, nchunk, 3, RB * Wq), f32),
    )(qp, wpar, bd1[None, :], w2d, bd2[:, None])

    out6 = out5.reshape(2, 2, B, nchunk, 3, RB, Wq)
    recon = out6.transpose(2, 4, 3, 5, 0, 6, 1).reshape(B, 3, H, W)
    return (recon, loss, idx)
